# Initial kernel scaffold; baseline (speedup 1.0000x reference)
import jax
import jax.numpy as jnp
from jax.experimental import pallas as pl


def kernel(x, grads):
    raise NotImplementedError("write your pallas kernel here")


# speedup vs baseline: 3.9099x; 3.9099x over previous
import jax
import jax.numpy as jnp
from jax import lax
from jax.experimental import pallas as pl
from jax.experimental.pallas import tpu as pltpu
from jax.experimental.pallas import tpu_sc as plsc

N_DIMS = 3
N_FIELDS = 64
RES = 32
TABLE_ROWS = (RES + 1) ** 3
ROW_LEN = N_FIELDS * N_DIMS
BATCH = 16384

L = 16
NW = 32
PTS_PER_W = BATCH // NW
CH = 16
N_CH = PTS_PER_W // CH
ROWS_PER_CH = CH * 8

_CORNERS = [(c0, c1, c2) for c0 in (0, 1) for c1 in (0, 1) for c2 in (0, 1)]


def _sc_body(table_hbm, xt_hbm, out_hbm, xv, idxv, coefv, rows, outv, sem):
    wid = lax.axis_index("s") * 2 + lax.axis_index("c")
    wbase = wid * PTS_PER_W

    pltpu.sync_copy(xt_hbm.at[:, pl.ds(wbase, PTS_PER_W)], xv)

    iota = lax.iota(jnp.int32, L)
    rowidx = [iota + j * CH for j in range(8)]

    def precompute(ci, _):
        xs = [xv[d, pl.ds(ci * L, L)] * float(RES) for d in range(N_DIMS)]
        ii = [x.astype(jnp.int32) for x in xs]
        ll = [xs[d] - ii[d].astype(jnp.float32) for d in range(N_DIMS)]
        base = ii[0] * ((RES + 1) * (RES + 1)) + ii[1] * (RES + 1) + ii[2]
        ss = [l * l * (3.0 - 2.0 * l) for l in ll]
        w0 = [1.0 - s for s in ss]
        for j, (c0, c1, c2) in enumerate(_CORNERS):
            off = c0 * (RES + 1) * (RES + 1) + c1 * (RES + 1) + c2
            idxv[ci, pl.ds(j * L, L)] = base + off
            wgt = (ss[0] if c0 else w0[0]) * (ss[1] if c1 else w0[1]) * (
                ss[2] if c2 else w0[2])
            for d, cd in enumerate((c0, c1, c2)):
                coefv[j * N_DIMS + d, pl.ds(ci * L, L)] = wgt * (ll[d] - float(cd))
        return 0

    lax.fori_loop(0, N_CH, precompute, 0)

    def chunk_body(ci, _):
        pltpu.async_copy(table_hbm.at[idxv.at[ci]], rows, sem).wait()
        cvecs = [coefv[s, pl.ds(ci * L, L)] for s in range(24)]

        def field_body(f, _):
            cbase = jnp.zeros((L,), jnp.int32) + 3 * f
            acc = jnp.zeros((L,), jnp.float32)
            for j in range(8):
                for d in range(N_DIMS):
                    g = plsc.load_gather(rows, [rowidx[j], cbase + d])
                    acc = acc + g * cvecs[j * N_DIMS + d]
            outv[f] = acc
            return 0

        lax.fori_loop(0, N_FIELDS, field_body, 0)
        pltpu.sync_copy(outv, out_hbm.at[wbase // CH + ci])
        return 0

    lax.fori_loop(0, N_CH, chunk_body, 0)


@jax.jit
def kernel(x, grads):
    table = grads.reshape(TABLE_ROWS, ROW_LEN)
    xt = x.T
    mesh = plsc.VectorSubcoreMesh(core_axis_name="c", subcore_axis_name="s")
    f = pl.kernel(
        _sc_body,
        mesh=mesh,
        out_type=jax.ShapeDtypeStruct((BATCH // CH, N_FIELDS, CH), jnp.float32),
        scratch_types=[
            pltpu.VMEM((N_DIMS, PTS_PER_W), jnp.float32),
            pltpu.VMEM((N_CH, ROWS_PER_CH), jnp.int32),
            pltpu.VMEM((24, PTS_PER_W), jnp.float32),
            pltpu.VMEM((ROWS_PER_CH, ROW_LEN), jnp.float32),
            pltpu.VMEM((N_FIELDS, CH), jnp.float32),
            pltpu.SemaphoreType.DMA,
        ],
        compiler_params=pltpu.CompilerParams(
            use_tc_tiling_on_sc=False, needs_layout_passes=False),
    )
    out3 = f(table, xt)
    return out3.transpose(0, 2, 1).reshape(BATCH, N_FIELDS)


# speedup vs baseline: 5.8267x; 1.4902x over previous
import jax
import jax.numpy as jnp
from jax import lax
from jax.experimental import pallas as pl
from jax.experimental.pallas import tpu as pltpu
from jax.experimental.pallas import tpu_sc as plsc

N_DIMS = 3
N_FIELDS = 64
RES = 32
TABLE_ROWS = (RES + 1) ** 3
ROW_LEN = N_FIELDS * N_DIMS
BATCH = 16384

L = 16
NW = 32
PTS_PER_W = BATCH // NW
CH = 16
N_CH = PTS_PER_W // CH
ROWS_PER_CH = CH * 8

_CORNERS = [(c0, c1, c2) for c0 in (0, 1) for c1 in (0, 1) for c2 in (0, 1)]


def _sc_body(table_hbm, xt_hbm, out_hbm, xv, idxv, coefv, rows, outv, sem):
    wid = lax.axis_index("s") * 2 + lax.axis_index("c")
    wbase = wid * PTS_PER_W

    pltpu.sync_copy(xt_hbm.at[:, pl.ds(wbase, PTS_PER_W)], xv)

    iota = lax.iota(jnp.int32, L)
    cols = [[iota * 3 + (48 * k + d) for d in range(N_DIMS)] for k in range(4)]

    def precompute(ci, _):
        xs = [xv[d, pl.ds(ci * L, L)] * float(RES) for d in range(N_DIMS)]
        ii = [x.astype(jnp.int32) for x in xs]
        ll = [xs[d] - ii[d].astype(jnp.float32) for d in range(N_DIMS)]
        base = ii[0] * ((RES + 1) * (RES + 1)) + ii[1] * (RES + 1) + ii[2]
        ss = [l * l * (3.0 - 2.0 * l) for l in ll]
        w0 = [1.0 - s for s in ss]
        for j, (c0, c1, c2) in enumerate(_CORNERS):
            off = c0 * (RES + 1) * (RES + 1) + c1 * (RES + 1) + c2
            idxv[ci, pl.ds(j * L, L)] = base + off
            wgt = (ss[0] if c0 else w0[0]) * (ss[1] if c1 else w0[1]) * (
                ss[2] if c2 else w0[2])
            for d, cd in enumerate((c0, c1, c2)):
                coefv[j * N_DIMS + d, pl.ds(ci * L, L)] = wgt * (ll[d] - float(cd))
        return 0

    lax.fori_loop(0, N_CH, precompute, 0)

    def chunk_body(ci, _):
        pltpu.async_copy(table_hbm.at[idxv.at[ci]], rows, sem).wait()
        cvecs = [coefv[s, pl.ds(ci * L, L)] for s in range(24)]
        for p in range(CH):
            acc = [jnp.zeros((L,), jnp.float32) for _ in range(4)]
            for j in range(8):
                rsp = jnp.full((L,), j * CH + p, jnp.int32)
                for d in range(N_DIMS):
                    cf = cvecs[j * N_DIMS + d][p]
                    for k in range(4):
                        g = plsc.load_gather(rows, [rsp, cols[k][d]])
                        acc[k] = acc[k] + g * cf
            for k in range(4):
                outv[p, pl.ds(k * L, L)] = acc[k]
        pltpu.sync_copy(outv, out_hbm.at[wbase // CH + ci])
        return 0

    lax.fori_loop(0, N_CH, chunk_body, 0)


@jax.jit
def kernel(x, grads):
    table = grads.reshape(TABLE_ROWS, ROW_LEN)
    xt = x.T
    mesh = plsc.VectorSubcoreMesh(core_axis_name="c", subcore_axis_name="s")
    f = pl.kernel(
        _sc_body,
        mesh=mesh,
        out_type=jax.ShapeDtypeStruct((BATCH // CH, CH, N_FIELDS), jnp.float32),
        scratch_types=[
            pltpu.VMEM((N_DIMS, PTS_PER_W), jnp.float32),
            pltpu.VMEM((N_CH, ROWS_PER_CH), jnp.int32),
            pltpu.VMEM((24, PTS_PER_W), jnp.float32),
            pltpu.VMEM((ROWS_PER_CH, ROW_LEN), jnp.float32),
            pltpu.VMEM((CH, N_FIELDS), jnp.float32),
            pltpu.SemaphoreType.DMA,
        ],
        compiler_params=pltpu.CompilerParams(
            use_tc_tiling_on_sc=False, needs_layout_passes=False),
    )
    out3 = f(table, xt)
    return out3.reshape(BATCH, N_FIELDS)


# speedup vs baseline: 6.1538x; 1.0561x over previous
import jax
import jax.numpy as jnp
from jax import lax
from jax.experimental import pallas as pl
from jax.experimental.pallas import tpu as pltpu
from jax.experimental.pallas import tpu_sc as plsc

N_DIMS = 3
N_FIELDS = 64
RES = 32
TABLE_ROWS = (RES + 1) ** 3
ROW_LEN = N_FIELDS * N_DIMS
BATCH = 16384

L = 16
NW = 32
PTS_PER_W = BATCH // NW
CH = 16
N_CH = PTS_PER_W // CH
ROWS_PER_CH = CH * 8

_CORNERS = [(c0, c1, c2) for c0 in (0, 1) for c1 in (0, 1) for c2 in (0, 1)]


def _sc_body(a_hbm, b_hbm, x_hbm, out_hbm,
             xv, idxv, coefv, rowsA, rowsB, outv, semA, semB, semO):
    wid = lax.axis_index("s") * 2 + lax.axis_index("c")
    wbase = wid * PTS_PER_W

    pltpu.sync_copy(x_hbm.at[pl.ds(wbase * N_DIMS, PTS_PER_W * N_DIMS)], xv)

    iota = lax.iota(jnp.int32, L)
    cols = [[iota * 3 + (48 * (k % 2) + d) for d in range(N_DIMS)]
            for k in range(4)]

    def precompute(ci, _):
        xs = [plsc.load_gather(xv, [iota * N_DIMS + (ci * (L * N_DIMS) + d)])
              * float(RES) for d in range(N_DIMS)]
        ii = [x.astype(jnp.int32) for x in xs]
        ll = [xs[d] - ii[d].astype(jnp.float32) for d in range(N_DIMS)]
        iw = [jnp.bitwise_and(i, RES - 1) for i in ii]
        base = iw[0] * ((RES + 1) * (RES + 1)) + iw[1] * (RES + 1) + iw[2]
        ss = [l * l * (3.0 - 2.0 * l) for l in ll]
        w0 = [1.0 - s for s in ss]
        for j, (c0, c1, c2) in enumerate(_CORNERS):
            off = c0 * (RES + 1) * (RES + 1) + c1 * (RES + 1) + c2
            idxv[ci, pl.ds(j * L, L)] = base + off
            wgt = (ss[0] if c0 else w0[0]) * (ss[1] if c1 else w0[1]) * (
                ss[2] if c2 else w0[2])
            for d, cd in enumerate((c0, c1, c2)):
                coefv[j * N_DIMS + d, pl.ds(ci * L, L)] = wgt * (ll[d] - float(cd))
        return 0

    lax.fori_loop(0, N_CH, precompute, 0)

    def fire_gather(ci, par):
        pltpu.async_copy(a_hbm.at[idxv.at[ci]], rowsA.at[par], semA.at[par])
        pltpu.async_copy(b_hbm.at[idxv.at[ci]], rowsB.at[par], semB.at[par])

    def out_slice(ci):
        return out_hbm.at[pl.ds((wbase // CH + ci) * 8, 8)]

    fire_gather(0, 0)
    fire_gather(1, 1)

    def chunk_body(ci, _):
        par = lax.rem(ci, 2)
        @pl.when(ci >= 2)
        def _():
            pltpu.make_async_copy(outv.at[par], out_slice(ci - 2),
                                  semO.at[par]).wait()
        pltpu.make_async_copy(a_hbm.at[idxv.at[ci]], rowsA.at[par],
                              semA.at[par]).wait()
        pltpu.make_async_copy(b_hbm.at[idxv.at[ci]], rowsB.at[par],
                              semB.at[par]).wait()
        cvecs = [coefv[s, pl.ds(ci * L, L)] for s in range(24)]
        for p in range(CH):
            acc = [jnp.zeros((L,), jnp.float32) for _ in range(4)]
            for j in range(8):
                rsp = jnp.full((L,), j * CH + p, jnp.int32)
                for d in range(N_DIMS):
                    cf = cvecs[j * N_DIMS + d][p]
                    for k in range(4):
                        half = rowsA if k < 2 else rowsB
                        g = plsc.load_gather(half.at[par], [rsp, cols[k][d]])
                        acc[k] = acc[k] + g * cf
            for k in range(4):
                outv[par, p // 2, pl.ds((p % 2) * N_FIELDS + k * L, L)] = acc[k]
        @pl.when(ci + 2 < N_CH)
        def _():
            fire_gather(ci + 2, par)
        pltpu.async_copy(outv.at[par], out_slice(ci), semO.at[par])
        return 0

    lax.fori_loop(0, N_CH, chunk_body, 0)

    pltpu.make_async_copy(outv.at[0], out_slice(N_CH - 2), semO.at[0]).wait()
    pltpu.make_async_copy(outv.at[1], out_slice(N_CH - 1), semO.at[1]).wait()


@jax.jit
def kernel(x, grads):
    a = jnp.pad(grads[:, :, :, :32, :].reshape(TABLE_ROWS, 96),
                ((0, 0), (0, 32)))
    b = jnp.pad(grads[:, :, :, 32:, :].reshape(TABLE_ROWS, 96),
                ((0, 0), (0, 32)))
    x1 = x.reshape(BATCH * N_DIMS)
    mesh = plsc.VectorSubcoreMesh(core_axis_name="c", subcore_axis_name="s")
    f = pl.kernel(
        _sc_body,
        mesh=mesh,
        out_type=jax.ShapeDtypeStruct((BATCH * N_FIELDS // 128, 128),
                                      jnp.float32),
        scratch_types=[
            pltpu.VMEM((PTS_PER_W * N_DIMS,), jnp.float32),
            pltpu.VMEM((N_CH, ROWS_PER_CH), jnp.int32),
            pltpu.VMEM((24, PTS_PER_W), jnp.float32),
            pltpu.VMEM((2, ROWS_PER_CH, 128), jnp.float32),
            pltpu.VMEM((2, ROWS_PER_CH, 128), jnp.float32),
            pltpu.VMEM((2, 8, 128), jnp.float32),
            pltpu.SemaphoreType.DMA((2,)),
            pltpu.SemaphoreType.DMA((2,)),
            pltpu.SemaphoreType.DMA((2,)),
        ],
        compiler_params=pltpu.CompilerParams(
            use_tc_tiling_on_sc=False, needs_layout_passes=False),
    )
    out2 = f(a, b, x1)
    return out2.reshape(BATCH, N_FIELDS)
